# baseline (speedup 1.0000x reference)
import functools

import jax
import jax.numpy as jnp
from jax import lax
from jax.experimental import pallas as pl
from jax.experimental.pallas import tpu as pltpu
from jax.experimental.pallas import tpu_sc as plsc

_BSZ, _SEQ, _D = 4, 8192, 2048
_NC, _NS, _L = 2, 16, 16
_NW = _NC * _NS
_ROWS_PER_W = _SEQ // _NW
_R = 8
_CHUNK = _R * _D
_N_CHUNKS = _ROWS_PER_W // _R
_UNROLL = 16


def _mul_chunk(x_v, p_v):
    def body(j, carry):
        base = j * (_L * _UNROLL)
        for u in range(_UNROLL):
            sl = pl.ds(base + u * _L, _L)
            x_v[sl] = x_v[sl] * p_v[sl]
        return carry

    lax.fori_loop(0, _CHUNK // (_L * _UNROLL), body, 0)


def _sc_body(x_hbm, p_hbm, o_hbm, x_v, p_v):
    wid = lax.axis_index("s") * _NC + lax.axis_index("c")

    def chunk_body(i, carry):
        off_p = (wid * _ROWS_PER_W + i * _R) * _D
        pltpu.sync_copy(p_hbm.at[pl.ds(off_p, _CHUNK)], p_v)

        def batch_body(b, carry2):
            off = b * (_SEQ * _D) + off_p
            pltpu.sync_copy(x_hbm.at[pl.ds(off, _CHUNK)], x_v)
            _mul_chunk(x_v, p_v)
            pltpu.sync_copy(x_v, o_hbm.at[pl.ds(off, _CHUNK)])
            return carry2

        lax.fori_loop(0, _BSZ, batch_body, 0)
        return carry

    lax.fori_loop(0, _N_CHUNKS, chunk_body, 0)


def kernel(input_tensor, position_vectors):
    bsz, seq_len, d = input_tensor.shape
    x = input_tensor.reshape(bsz * seq_len * d)
    p = position_vectors[:seq_len, :d].reshape(seq_len * d)
    run = pl.kernel(
        _sc_body,
        out_type=jax.ShapeDtypeStruct((bsz * seq_len * d,), input_tensor.dtype),
        mesh=plsc.VectorSubcoreMesh(core_axis_name="c", subcore_axis_name="s"),
        scratch_types=[
            pltpu.VMEM((_CHUNK,), jnp.float32),
            pltpu.VMEM((_CHUNK,), jnp.float32),
        ],
    )
    return run(x, p).reshape(bsz, seq_len, d)


# speedup vs baseline: 1.3278x; 1.3278x over previous
import jax
import jax.numpy as jnp
from jax import lax
from jax.experimental import pallas as pl
from jax.experimental.pallas import tpu as pltpu
from jax.experimental.pallas import tpu_sc as plsc

_BSZ, _SEQ, _D = 4, 8192, 2048
_NC, _NS, _L = 2, 16, 16
_NW = _NC * _NS
_ROWS_PER_W = _SEQ // _NW
_R = 8
_CHUNK = _R * _D
_N_CHUNKS = _ROWS_PER_W // _R
_N_STEPS = _N_CHUNKS * _BSZ
_UNROLL = 16


def _mul_chunk(o_v, x_v, p_v):
    def body(j, carry):
        base = j * (_L * _UNROLL)
        for u in range(_UNROLL):
            sl = pl.ds(base + u * _L, _L)
            o_v[sl] = x_v[sl] * p_v[sl]
        return carry

    lax.fori_loop(0, _CHUNK // (_L * _UNROLL), body, 0)


def _sc_body(x_hbm, p_hbm, o_hbm,
             x0, x1, o0, o1, p0, p1,
             xs0, xs1, os0, os1, ps0, ps1):
    x_bufs, o_bufs, p_bufs = (x0, x1), (o0, o1), (p0, p1)
    x_sems, o_sems, p_sems = (xs0, xs1), (os0, os1), (ps0, ps1)
    wid = lax.axis_index("s") * _NC + lax.axis_index("c")
    row0 = wid * _ROWS_PER_W

    def p_off(i):
        return (row0 + i * _R) * _D

    def x_off(i, b):
        return b * (_SEQ * _D) + p_off(i)

    for k in range(2):
        pltpu.make_async_copy(
            p_hbm.at[pl.ds(p_off(k), _CHUNK)], p_bufs[k], p_sems[k]).start()
        pltpu.make_async_copy(
            x_hbm.at[pl.ds(x_off(0, k), _CHUNK)], x_bufs[k], x_sems[k]).start()

    def chunk_body(i2, carry):
        for ip in range(2):
            i = i2 * 2 + ip
            for b in range(_BSZ):
                k = b % 2
                g = i * _BSZ + b
                pltpu.make_async_copy(
                    x_hbm.at[pl.ds(0, _CHUNK)], x_bufs[k], x_sems[k]).wait()
                if b == 0:
                    pltpu.make_async_copy(
                        p_hbm.at[pl.ds(0, _CHUNK)], p_bufs[ip], p_sems[ip]).wait()

                @pl.when(g >= 2)
                def _():
                    pltpu.make_async_copy(
                        o_bufs[k], o_hbm.at[pl.ds(0, _CHUNK)], o_sems[k]).wait()

                _mul_chunk(o_bufs[k], x_bufs[k], p_bufs[ip])

                pltpu.make_async_copy(
                    o_bufs[k], o_hbm.at[pl.ds(x_off(i, b), _CHUNK)],
                    o_sems[k]).start()

                i_n = i + (b + 2) // _BSZ
                b_n = (b + 2) % _BSZ

                @pl.when(g + 2 < _N_STEPS)
                def _():
                    pltpu.make_async_copy(
                        x_hbm.at[pl.ds(x_off(i_n, b_n), _CHUNK)],
                        x_bufs[k], x_sems[k]).start()

            @pl.when(i + 2 < _N_CHUNKS)
            def _():
                pltpu.make_async_copy(
                    p_hbm.at[pl.ds(p_off(i + 2), _CHUNK)],
                    p_bufs[ip], p_sems[ip]).start()
        return carry

    lax.fori_loop(0, _N_CHUNKS // 2, chunk_body, 0)

    for k in range(2):
        pltpu.make_async_copy(
            o_bufs[k], o_hbm.at[pl.ds(0, _CHUNK)], o_sems[k]).wait()


def kernel(input_tensor, position_vectors):
    bsz, seq_len, d = input_tensor.shape
    x = input_tensor.reshape(bsz * seq_len * d)
    p = position_vectors[:seq_len, :d].reshape(seq_len * d)
    run = pl.kernel(
        _sc_body,
        out_type=jax.ShapeDtypeStruct((bsz * seq_len * d,), input_tensor.dtype),
        mesh=plsc.VectorSubcoreMesh(core_axis_name="c", subcore_axis_name="s"),
        scratch_types=(
            [pltpu.VMEM((_CHUNK,), jnp.float32) for _ in range(6)]
            + [pltpu.SemaphoreType.DMA for _ in range(6)]
        ),
    )
    return run(x, p).reshape(bsz, seq_len, d)


# speedup vs baseline: 1.3307x; 1.0022x over previous
import jax
import jax.numpy as jnp
from jax import lax
from jax.experimental import pallas as pl
from jax.experimental.pallas import tpu as pltpu
from jax.experimental.pallas import tpu_sc as plsc

_BSZ, _SEQ, _D = 4, 8192, 2048
_NC, _NS, _L = 2, 16, 16
_NW = _NC * _NS
_ROWS_PER_W = _SEQ // _NW
_R = 8
_CHUNK = _R * _D
_N_CHUNKS = _ROWS_PER_W // _R
_N_STEPS = _N_CHUNKS * _BSZ
_UNROLL = 16


def _mul_chunk(o_v, x_v, p_v):
    @plsc.parallel_loop(0, _CHUNK, _L, unroll=_UNROLL)
    def body(i):
        sl = pl.ds(i, _L)
        o_v[sl] = x_v[sl] * p_v[sl]


def _sc_body(x_hbm, p_hbm, o_hbm,
             x0, x1, o0, o1, p0, p1,
             xs0, xs1, os0, os1, ps0, ps1):
    x_bufs, o_bufs, p_bufs = (x0, x1), (o0, o1), (p0, p1)
    x_sems, o_sems, p_sems = (xs0, xs1), (os0, os1), (ps0, ps1)
    wid = lax.axis_index("s") * _NC + lax.axis_index("c")
    row0 = wid * _ROWS_PER_W

    def p_off(i):
        return (row0 + i * _R) * _D

    def x_off(i, b):
        return b * (_SEQ * _D) + p_off(i)

    for k in range(2):
        pltpu.make_async_copy(
            p_hbm.at[pl.ds(p_off(k), _CHUNK)], p_bufs[k], p_sems[k]).start()
        pltpu.make_async_copy(
            x_hbm.at[pl.ds(x_off(0, k), _CHUNK)], x_bufs[k], x_sems[k]).start()

    def chunk_body(i2, carry):
        for ip in range(2):
            i = i2 * 2 + ip
            for b in range(_BSZ):
                k = b % 2
                g = i * _BSZ + b
                pltpu.make_async_copy(
                    x_hbm.at[pl.ds(0, _CHUNK)], x_bufs[k], x_sems[k]).wait()
                if b == 0:
                    pltpu.make_async_copy(
                        p_hbm.at[pl.ds(0, _CHUNK)], p_bufs[ip], p_sems[ip]).wait()

                @pl.when(g >= 2)
                def _():
                    pltpu.make_async_copy(
                        o_bufs[k], o_hbm.at[pl.ds(0, _CHUNK)], o_sems[k]).wait()

                _mul_chunk(o_bufs[k], x_bufs[k], p_bufs[ip])

                pltpu.make_async_copy(
                    o_bufs[k], o_hbm.at[pl.ds(x_off(i, b), _CHUNK)],
                    o_sems[k]).start()

                i_n = i + (b + 2) // _BSZ
                b_n = (b + 2) % _BSZ

                @pl.when(g + 2 < _N_STEPS)
                def _():
                    pltpu.make_async_copy(
                        x_hbm.at[pl.ds(x_off(i_n, b_n), _CHUNK)],
                        x_bufs[k], x_sems[k]).start()

            @pl.when(i + 2 < _N_CHUNKS)
            def _():
                pltpu.make_async_copy(
                    p_hbm.at[pl.ds(p_off(i + 2), _CHUNK)],
                    p_bufs[ip], p_sems[ip]).start()
        return carry

    lax.fori_loop(0, _N_CHUNKS // 2, chunk_body, 0)

    for k in range(2):
        pltpu.make_async_copy(
            o_bufs[k], o_hbm.at[pl.ds(0, _CHUNK)], o_sems[k]).wait()


def kernel(input_tensor, position_vectors):
    bsz, seq_len, d = input_tensor.shape
    x = input_tensor.reshape(bsz * seq_len * d)
    p = position_vectors[:seq_len, :d].reshape(seq_len * d)
    run = pl.kernel(
        _sc_body,
        out_type=jax.ShapeDtypeStruct((bsz * seq_len * d,), input_tensor.dtype),
        mesh=plsc.VectorSubcoreMesh(core_axis_name="c", subcore_axis_name="s"),
        scratch_types=(
            [pltpu.VMEM((_CHUNK,), jnp.float32) for _ in range(6)]
            + [pltpu.SemaphoreType.DMA for _ in range(6)]
        ),
    )
    return run(x, p).reshape(bsz, seq_len, d)


# speedup vs baseline: 5.3485x; 4.0193x over previous
import jax
import jax.numpy as jnp
from jax.experimental import pallas as pl

_S_BLK = 1024


def _bind_kernel(x_ref, p_ref, o_ref):
    o_ref[...] = x_ref[...] * p_ref[...]


def kernel(input_tensor, position_vectors):
    bsz, seq_len, d = input_tensor.shape
    pos = position_vectors[:seq_len, :d]
    grid = (seq_len // _S_BLK, bsz)
    return pl.pallas_call(
        _bind_kernel,
        grid=grid,
        in_specs=[
            pl.BlockSpec((1, _S_BLK, d), lambda s, b: (b, s, 0)),
            pl.BlockSpec((_S_BLK, d), lambda s, b: (s, 0)),
        ],
        out_specs=pl.BlockSpec((1, _S_BLK, d), lambda s, b: (b, s, 0)),
        out_shape=jax.ShapeDtypeStruct((bsz, seq_len, d), input_tensor.dtype),
    )(input_tensor, pos)
